# Initial kernel scaffold; baseline (speedup 1.0000x reference)
import jax
import jax.numpy as jnp
from jax.experimental import pallas as pl


def kernel(input, edge_index):
    raise NotImplementedError("write your pallas kernel here")


# speedup vs baseline: 5.6414x; 5.6414x over previous
import jax
import jax.numpy as jnp
from jax import lax
from jax.experimental import pallas as pl
from jax.experimental.pallas import tpu as pltpu
from jax.experimental.pallas import tpu_sc as plsc

N = 10000
D = 128
E = 320000
EPS = 1e-15
BALL_EPS = 4e-3
DPAD = 144
NW = 32
EPW = E // NW
B = 16
NB = EPW // B
NACC = 10240
RPS = NACC // 16
ZROWS = 128


def _artanh(v):
    v = jnp.clip(v, -1.0 + 1e-7, 1.0 - 1e-7)
    return 0.5 * (jnp.log1p(v) - jnp.log1p(-v))


def _node_stats_body(x_ref, x2_ref, lm_ref):
    x = x_ref[...]
    x2 = jnp.sum(x * x, axis=1, keepdims=True)
    nrm = jnp.maximum(jnp.sqrt(x2), EPS)
    lm_ref[...] = x / nrm * _artanh(nrm)
    x2_ref[...] = x2


_node_stats = pl.pallas_call(
    _node_stats_body,
    out_shape=(
        jax.ShapeDtypeStruct((N, 1), jnp.float32),
        jax.ShapeDtypeStruct((N, D), jnp.float32),
    ),
)


def _sc_dots_body(x_hbm, x2_hbm, src_hbm, dst_hbm, out_hbm,
                  srcb, dstb, x2sb, x2db, m2b, rows_s, rows_d,
                  sem_s, sem_d, sem_a, sem_b):
    c = lax.axis_index("c")
    s = lax.axis_index("s")
    w = s * 2 + c
    base = pl.multiple_of(w * EPW, EPW)
    pltpu.sync_copy(src_hbm.at[pl.ds(base, EPW)], srcb)
    pltpu.sync_copy(dst_hbm.at[pl.ds(base, EPW)], dstb)

    def step(i, carry):
        off = pl.multiple_of(i * B, B)
        cp1 = pltpu.async_copy(x_hbm.at[srcb.at[pl.ds(off, B)]], rows_s, sem_s)
        cp2 = pltpu.async_copy(x_hbm.at[dstb.at[pl.ds(off, B)]], rows_d, sem_d)
        cp3 = pltpu.async_copy(x2_hbm.at[srcb.at[pl.ds(off, B)]], x2sb, sem_a)
        cp4 = pltpu.async_copy(x2_hbm.at[dstb.at[pl.ds(off, B)]], x2db, sem_b)
        cp1.wait()
        cp2.wait()
        cp3.wait()
        cp4.wait()
        lanes = lax.iota(jnp.int32, 16)
        perms = [jnp.bitwise_xor(lanes, sh) for sh in (8, 4, 2, 1)]
        xy = jnp.zeros((16,), jnp.float32)
        for e in range(B):
            acc = rows_s[e, pl.ds(0, 16)] * rows_d[e, pl.ds(0, 16)]
            for cc in range(1, 8):
                acc = acc + rows_s[e, pl.ds(cc * 16, 16)] * rows_d[e, pl.ds(cc * 16, 16)]
            for p in perms:
                acc = acc + acc.at[p].get(mode="promise_in_bounds")
            xy = jnp.where(lanes == e, acc, xy)
        x2s = x2sb[...]
        x2d = x2db[...]
        a1 = 1.0 - 2.0 * xy + x2d
        b1 = 1.0 - x2s
        den = jnp.maximum(1.0 - 2.0 * xy + x2s * x2d, EPS)
        num2 = jnp.maximum(a1 * a1 * x2s - 2.0 * a1 * b1 * xy + b1 * b1 * x2d, 0.0)
        m2b[pl.ds(off, B)] = num2 / (den * den)
        return carry

    lax.fori_loop(0, NB, step, 0)
    pltpu.sync_copy(m2b, out_hbm.at[w])


_sc_dots = pl.kernel(
    _sc_dots_body,
    out_type=jax.ShapeDtypeStruct((NW, EPW), jnp.float32),
    mesh=plsc.VectorSubcoreMesh(core_axis_name="c", subcore_axis_name="s"),
    scratch_types=[
        pltpu.VMEM((EPW,), jnp.int32),
        pltpu.VMEM((EPW,), jnp.int32),
        pltpu.VMEM((B,), jnp.float32),
        pltpu.VMEM((B,), jnp.float32),
        pltpu.VMEM((EPW,), jnp.float32),
        pltpu.VMEM((B, D), jnp.float32),
        pltpu.VMEM((B, D), jnp.float32),
        pltpu.SemaphoreType.DMA,
        pltpu.SemaphoreType.DMA,
        pltpu.SemaphoreType.DMA,
        pltpu.SemaphoreType.DMA,
    ],
)


def _edge_t_body(m2_ref, t_ref):
    m2 = jnp.maximum(m2_ref[...], 0.0)
    dd = 2.0 * _artanh(jnp.sqrt(m2))
    t_ref[...] = jnp.exp(dd * dd)


_edge_t = pl.pallas_call(
    _edge_t_body,
    out_shape=jax.ShapeDtypeStruct((E // 128, 128), jnp.float32),
)


def _sc_agg_body(lm_hbm, src_hbm, dst_hbm, t_hbm, out2d_hbm,
                 srcb, dstb, tb, rows, zbuf, acc, sem):
    c = lax.axis_index("c")
    s = lax.axis_index("s")
    w = s * 2 + c
    base = pl.multiple_of(w * EPW, EPW)

    zv = jnp.zeros((16,), jnp.float32)

    def zrow(r, carry):
        for cc in range(D // 16):
            zbuf[r, pl.ds(cc * 16, 16)] = zv
        return carry

    lax.fori_loop(0, ZROWS, zrow, 0)
    for k in range(RPS // ZROWS):
        pltpu.sync_copy(zbuf, acc.at[pl.ds(s * RPS + k * ZROWS, ZROWS)])
    plsc.subcore_barrier()

    pltpu.sync_copy(src_hbm.at[pl.ds(base, EPW)], srcb)
    pltpu.sync_copy(dst_hbm.at[pl.ds(base, EPW)], dstb)
    pltpu.sync_copy(t_hbm.at[pl.ds(base, EPW)], tb)

    def step(i, carry):
        off = pl.multiple_of(i * B, B)
        pltpu.async_copy(lm_hbm.at[dstb.at[pl.ds(off, B)]], rows, sem).wait()
        tvec = tb[pl.ds(off, B)]
        for e in range(B):
            tv = tvec[e]
            for cc in range(D // 16):
                rows[e, pl.ds(cc * 16, 16)] = rows[e, pl.ds(cc * 16, 16)] * tv
        sv = srcb[pl.ds(off, B)]
        pltpu.sync_copy(rows, acc.at[sv], add=True)
        return carry

    lax.fori_loop(0, NB, step, 0)
    plsc.subcore_barrier()
    pltpu.sync_copy(acc.at[pl.ds(s * RPS, RPS)], out2d_hbm.at[c, pl.ds(s * RPS, RPS)])


_sc_agg = pl.kernel(
    _sc_agg_body,
    out_type=jax.ShapeDtypeStruct((2, NACC, D), jnp.float32),
    mesh=plsc.VectorSubcoreMesh(core_axis_name="c", subcore_axis_name="s"),
    scratch_types=[
        pltpu.VMEM((EPW,), jnp.int32),
        pltpu.VMEM((EPW,), jnp.int32),
        pltpu.VMEM((EPW,), jnp.float32),
        pltpu.VMEM((B, D), jnp.float32),
        pltpu.VMEM((ZROWS, D), jnp.float32),
        pltpu.VMEM_SHARED((NACC, D), jnp.float32),
        pltpu.SemaphoreType.DMA,
    ],
)


def _sc_segsum_body(src_hbm, t_hbm, outs_hbm, srcb, tb, zsbuf, accs, sem):
    c = lax.axis_index("c")
    s = lax.axis_index("s")
    w = s * 2 + c
    base = pl.multiple_of(w * EPW, EPW)

    zv = jnp.zeros((16,), jnp.float32)

    def zsrow(r, carry):
        zsbuf[pl.ds(r * 16, 16)] = zv
        return carry

    lax.fori_loop(0, RPS // 16, zsrow, 0)
    pltpu.sync_copy(zsbuf, accs.at[pl.ds(s * RPS, RPS)])
    plsc.subcore_barrier()

    pltpu.sync_copy(src_hbm.at[pl.ds(base, EPW)], srcb)
    pltpu.sync_copy(t_hbm.at[pl.ds(base, EPW)], tb)

    def step(i, carry):
        off = pl.multiple_of(i * B, B)
        sv = srcb[pl.ds(off, B)]
        pltpu.sync_copy(tb.at[pl.ds(off, B)], accs.at[sv], add=True)
        return carry

    lax.fori_loop(0, NB, step, 0)
    plsc.subcore_barrier()
    pltpu.sync_copy(accs.at[pl.ds(s * RPS, RPS)], outs_hbm.at[c, pl.ds(s * RPS, RPS)])


_sc_segsum = pl.kernel(
    _sc_segsum_body,
    out_type=jax.ShapeDtypeStruct((2, NACC), jnp.float32),
    mesh=plsc.VectorSubcoreMesh(core_axis_name="c", subcore_axis_name="s"),
    scratch_types=[
        pltpu.VMEM((EPW,), jnp.int32),
        pltpu.VMEM((EPW,), jnp.float32),
        pltpu.VMEM((RPS,), jnp.float32),
        pltpu.VMEM_SHARED((NACC,), jnp.float32),
        pltpu.SemaphoreType.DMA,
    ],
)


def _finalize_body(p_ref, ps_ref, o_ref):
    num = p_ref[0] + p_ref[1]
    seg = ps_ref[0] + ps_ref[1]
    y = num / (seg + EPS)
    n2 = jnp.maximum(jnp.sqrt(jnp.sum(y * y, axis=1, keepdims=True)), EPS)
    res = jnp.tanh(n2) * y / n2
    n3 = jnp.maximum(jnp.sqrt(jnp.sum(res * res, axis=1, keepdims=True)), EPS)
    maxn = 1.0 - BALL_EPS
    o_ref[...] = jnp.where(n3 > maxn, res / n3 * maxn, res)


_finalize = pl.pallas_call(
    _finalize_body,
    out_shape=jax.ShapeDtypeStruct((NACC, D), jnp.float32),
)


def kernel(input, edge_index):
    x = input.astype(jnp.float32)
    ei = edge_index.astype(jnp.int32)
    src = ei[0]
    dst = ei[1]
    x2, lm = _node_stats(x)
    x2f = x2.reshape(N)
    m2 = _sc_dots(x, x2f, src, dst).reshape(E)
    t = _edge_t(m2.reshape(E // 128, 128)).reshape(E)
    part = _sc_agg(lm, src, dst, t)
    part_s = _sc_segsum(src, t)
    return _finalize(part, part_s.reshape(2, NACC, 1))[:N]


# speedup vs baseline: 16.2017x; 2.8719x over previous
import jax
import jax.numpy as jnp
from jax import lax
from jax.experimental import pallas as pl
from jax.experimental.pallas import tpu as pltpu
from jax.experimental.pallas import tpu_sc as plsc

N = 10000
D = 128
E = 320000
EPS = 1e-15
BALL_EPS = 4e-3
DPAD = 144
NW = 32
EPW = E // NW
B = 80
NB = EPW // B
PAIRS = (NB - 1) // 2
GROUPS = B // 16
BSS = 16
NBSS = EPW // BSS
NACC = 10240
RPS = NACC // 16
ZROWS = 32


def _artanh(v):
    v = jnp.clip(v, -1.0 + 1e-7, 1.0 - 1e-7)
    return 0.5 * (jnp.log1p(v) - jnp.log1p(-v))


def _node_stats_body(x_ref, x2_ref, lm_ref):
    x = x_ref[...]
    x2 = jnp.sum(x * x, axis=1, keepdims=True)
    nrm = jnp.maximum(jnp.sqrt(x2), EPS)
    lm_ref[...] = x / nrm * _artanh(nrm)
    x2_ref[...] = x2


_node_stats = pl.pallas_call(
    _node_stats_body,
    out_shape=(
        jax.ShapeDtypeStruct((N, 1), jnp.float32),
        jax.ShapeDtypeStruct((N, D), jnp.float32),
    ),
)


def _sc_dots_body(x_hbm, x2_hbm, src_hbm, dst_hbm, out_hbm,
                  srcb, dstb, m2b,
                  rs0, rd0, xs0, xd0, rs1, rd1, xs1, xd1,
                  m_rs0, m_rd0, m_xs0, m_xd0, m_rs1, m_rd1, m_xs1, m_xd1):
    c = lax.axis_index("c")
    s = lax.axis_index("s")
    w = s * 2 + c
    base = pl.multiple_of(w * EPW, EPW)
    pltpu.sync_copy(src_hbm.at[pl.ds(base, EPW)], srcb)
    pltpu.sync_copy(dst_hbm.at[pl.ds(base, EPW)], dstb)

    bufs = (
        (rs0, rd0, xs0, xd0, m_rs0, m_rd0, m_xs0, m_xd0),
        (rs1, rd1, xs1, xd1, m_rs1, m_rd1, m_xs1, m_xd1),
    )

    def issue(i, bs):
        off = pl.multiple_of(i * B, B)
        pltpu.async_copy(x_hbm.at[srcb.at[pl.ds(off, B)]], bs[0], bs[4])
        pltpu.async_copy(x_hbm.at[dstb.at[pl.ds(off, B)]], bs[1], bs[5])
        pltpu.async_copy(x2_hbm.at[srcb.at[pl.ds(off, B)]], bs[2], bs[6])
        pltpu.async_copy(x2_hbm.at[dstb.at[pl.ds(off, B)]], bs[3], bs[7])

    def wait(bs):
        pltpu.make_async_copy(x_hbm.at[srcb.at[pl.ds(0, B)]], bs[0], bs[4]).wait()
        pltpu.make_async_copy(x_hbm.at[dstb.at[pl.ds(0, B)]], bs[1], bs[5]).wait()
        pltpu.make_async_copy(x2_hbm.at[srcb.at[pl.ds(0, B)]], bs[2], bs[6]).wait()
        pltpu.make_async_copy(x2_hbm.at[dstb.at[pl.ds(0, B)]], bs[3], bs[7]).wait()

    lanes = lax.iota(jnp.int32, 16)
    perms = [jnp.bitwise_xor(lanes, sh) for sh in (8, 4, 2, 1)]

    def compute(i, bs):
        rs, rd, xsb, xdb = bs[0], bs[1], bs[2], bs[3]
        off = pl.multiple_of(i * B, B)

        def group(g, carry):
            gb = pl.multiple_of(g * 16, 16)
            xy = jnp.zeros((16,), jnp.float32)
            for e in range(16):
                r = gb + e
                acc = rs[r, pl.ds(0, 16)] * rd[r, pl.ds(0, 16)]
                for cc in range(1, 8):
                    acc = acc + rs[r, pl.ds(cc * 16, 16)] * rd[r, pl.ds(cc * 16, 16)]
                for p in perms:
                    acc = acc + acc.at[p].get(mode="promise_in_bounds")
                xy = jnp.where(lanes == e, acc, xy)
            x2s = xsb[pl.ds(gb, 16)]
            x2d = xdb[pl.ds(gb, 16)]
            a1 = 1.0 - 2.0 * xy + x2d
            b1 = 1.0 - x2s
            den = jnp.maximum(1.0 - 2.0 * xy + x2s * x2d, EPS)
            num2 = jnp.maximum(a1 * a1 * x2s - 2.0 * a1 * b1 * xy + b1 * b1 * x2d, 0.0)
            m2b[pl.ds(pl.multiple_of(off + gb, 16), 16)] = num2 / (den * den)
            return carry

        lax.fori_loop(0, GROUPS, group, 0)

    issue(0, bufs[0])

    def pair(j, carry):
        i0 = j * 2
        wait(bufs[0])
        issue(i0 + 1, bufs[1])
        compute(i0, bufs[0])
        wait(bufs[1])
        issue(i0 + 2, bufs[0])
        compute(i0 + 1, bufs[1])
        return carry

    lax.fori_loop(0, PAIRS, pair, 0)
    wait(bufs[0])
    compute(NB - 1, bufs[0])
    pltpu.sync_copy(m2b, out_hbm.at[w])


_sc_dots = pl.kernel(
    _sc_dots_body,
    out_type=jax.ShapeDtypeStruct((NW, EPW), jnp.float32),
    mesh=plsc.VectorSubcoreMesh(core_axis_name="c", subcore_axis_name="s"),
    scratch_types=[
        pltpu.VMEM((EPW,), jnp.int32),
        pltpu.VMEM((EPW,), jnp.int32),
        pltpu.VMEM((EPW,), jnp.float32),
        pltpu.VMEM((B, D), jnp.float32),
        pltpu.VMEM((B, D), jnp.float32),
        pltpu.VMEM((B,), jnp.float32),
        pltpu.VMEM((B,), jnp.float32),
        pltpu.VMEM((B, D), jnp.float32),
        pltpu.VMEM((B, D), jnp.float32),
        pltpu.VMEM((B,), jnp.float32),
        pltpu.VMEM((B,), jnp.float32),
    ] + [pltpu.SemaphoreType.DMA] * 8,
)


def _edge_t_body(m2_ref, t_ref):
    m2 = jnp.maximum(m2_ref[...], 0.0)
    dd = 2.0 * _artanh(jnp.sqrt(m2))
    t_ref[...] = jnp.exp(dd * dd)


_edge_t = pl.pallas_call(
    _edge_t_body,
    out_shape=jax.ShapeDtypeStruct((E // 128, 128), jnp.float32),
)


def _sc_agg_body(lm_hbm, src_hbm, dst_hbm, t_hbm, out2d_hbm,
                 tb, didx0, didx1, sidx0, sidx1, rows0, rows1, zbuf, acc,
                 m_g0, m_g1, m_i0, m_i1, m_si0, m_si1):
    c = lax.axis_index("c")
    s = lax.axis_index("s")
    w = s * 2 + c
    base = pl.multiple_of(w * EPW, EPW)

    zv = jnp.zeros((16,), jnp.float32)

    def zrow(r, carry):
        for cc in range(D // 16):
            zbuf[r, pl.ds(cc * 16, 16)] = zv
        return carry

    lax.fori_loop(0, ZROWS, zrow, 0)
    for k in range(RPS // ZROWS):
        pltpu.sync_copy(zbuf, acc.at[pl.ds(s * RPS + k * ZROWS, ZROWS)])
    plsc.subcore_barrier()

    pltpu.sync_copy(t_hbm.at[pl.ds(base, EPW)], tb)

    didx = (didx0, didx1)
    sidx = (sidx0, sidx1)
    rows = (rows0, rows1)
    m_g = (m_g0, m_g1)
    m_i = (m_i0, m_i1)
    m_si = (m_si0, m_si1)

    def ixissue(i, b):
        off = pl.multiple_of(i * B, B)
        pltpu.async_copy(dst_hbm.at[pl.ds(base + off, B)], didx[b], m_i[b])

    def ixwait(b):
        pltpu.make_async_copy(dst_hbm.at[pl.ds(0, B)], didx[b], m_i[b]).wait()

    def sixissue(i, b):
        off = pl.multiple_of(i * B, B)
        pltpu.async_copy(src_hbm.at[pl.ds(base + off, B)], sidx[b], m_si[b])

    def sixwait(b):
        pltpu.make_async_copy(src_hbm.at[pl.ds(0, B)], sidx[b], m_si[b]).wait()

    def gissue(b):
        pltpu.async_copy(lm_hbm.at[didx[b]], rows[b], m_g[b])

    def gwait(b):
        pltpu.make_async_copy(lm_hbm.at[didx[b]], rows[b], m_g[b]).wait()

    def scale(i, b):
        off = pl.multiple_of(i * B, B)

        def group(g, carry):
            gb = pl.multiple_of(g * 16, 16)
            tvec = tb[pl.ds(pl.multiple_of(off + gb, 16), 16)]
            for e in range(16):
                tv = tvec[e]
                r = gb + e
                for cc in range(D // 16):
                    rows[b][r, pl.ds(cc * 16, 16)] = rows[b][r, pl.ds(cc * 16, 16)] * tv
            return carry

        lax.fori_loop(0, GROUPS, group, 0)

    pltpu.sync_copy(dst_hbm.at[pl.ds(base, B)], didx0)
    gissue(0)
    ixissue(1, 1)
    sixissue(0, 0)
    sixissue(1, 1)

    def substep(i, b):
        ixwait(1 - b)
        gissue(1 - b)
        gwait(b)
        ixissue(jnp.minimum(i + 2, NB - 1), b)
        scale(i, b)
        sixwait(b)
        pltpu.sync_copy(rows[b], acc.at[sidx[b]], add=True)
        sixissue(jnp.minimum(i + 2, NB - 1), b)

    def pair(j, carry):
        i0 = j * 2
        substep(i0, 0)
        substep(i0 + 1, 1)
        return carry

    lax.fori_loop(0, PAIRS, pair, 0)
    gwait(0)
    scale(NB - 1, 0)
    sixwait(0)
    pltpu.sync_copy(rows[0], acc.at[sidx[0]], add=True)
    ixwait(1)
    sixwait(1)
    plsc.subcore_barrier()
    pltpu.sync_copy(acc.at[pl.ds(s * RPS, RPS)], out2d_hbm.at[c, pl.ds(s * RPS, RPS)])


_sc_agg = pl.kernel(
    _sc_agg_body,
    out_type=jax.ShapeDtypeStruct((2, NACC, D), jnp.float32),
    mesh=plsc.VectorSubcoreMesh(core_axis_name="c", subcore_axis_name="s"),
    scratch_types=[
        pltpu.VMEM((EPW,), jnp.float32),
        pltpu.VMEM((B,), jnp.int32),
        pltpu.VMEM((B,), jnp.int32),
        pltpu.VMEM((B,), jnp.int32),
        pltpu.VMEM((B,), jnp.int32),
        pltpu.VMEM((B, D), jnp.float32),
        pltpu.VMEM((B, D), jnp.float32),
        pltpu.VMEM((ZROWS, D), jnp.float32),
        pltpu.VMEM_SHARED((NACC, D), jnp.float32),
    ] + [pltpu.SemaphoreType.DMA] * 6,
)


def _sc_segsum_body(src_hbm, t_hbm, outs_hbm, srcb, tb, zsbuf, accs, sem):
    c = lax.axis_index("c")
    s = lax.axis_index("s")
    w = s * 2 + c
    base = pl.multiple_of(w * EPW, EPW)

    zv = jnp.zeros((16,), jnp.float32)

    def zsrow(r, carry):
        zsbuf[pl.ds(r * 16, 16)] = zv
        return carry

    lax.fori_loop(0, RPS // 16, zsrow, 0)
    pltpu.sync_copy(zsbuf, accs.at[pl.ds(s * RPS, RPS)])
    plsc.subcore_barrier()

    pltpu.sync_copy(src_hbm.at[pl.ds(base, EPW)], srcb)
    pltpu.sync_copy(t_hbm.at[pl.ds(base, EPW)], tb)

    def step(i, carry):
        off = pl.multiple_of(i * BSS, BSS)
        sv = srcb[pl.ds(off, BSS)]
        pltpu.sync_copy(tb.at[pl.ds(off, BSS)], accs.at[sv], add=True)
        return carry

    lax.fori_loop(0, NBSS, step, 0)
    plsc.subcore_barrier()
    pltpu.sync_copy(accs.at[pl.ds(s * RPS, RPS)], outs_hbm.at[c, pl.ds(s * RPS, RPS)])


_sc_segsum = pl.kernel(
    _sc_segsum_body,
    out_type=jax.ShapeDtypeStruct((2, NACC), jnp.float32),
    mesh=plsc.VectorSubcoreMesh(core_axis_name="c", subcore_axis_name="s"),
    scratch_types=[
        pltpu.VMEM((EPW,), jnp.int32),
        pltpu.VMEM((EPW,), jnp.float32),
        pltpu.VMEM((RPS,), jnp.float32),
        pltpu.VMEM_SHARED((NACC,), jnp.float32),
        pltpu.SemaphoreType.DMA,
    ],
)


def _finalize_body(p_ref, ps_ref, o_ref):
    num = p_ref[0] + p_ref[1]
    seg = ps_ref[0] + ps_ref[1]
    y = num / (seg + EPS)
    n2 = jnp.maximum(jnp.sqrt(jnp.sum(y * y, axis=1, keepdims=True)), EPS)
    res = jnp.tanh(n2) * y / n2
    n3 = jnp.maximum(jnp.sqrt(jnp.sum(res * res, axis=1, keepdims=True)), EPS)
    maxn = 1.0 - BALL_EPS
    o_ref[...] = jnp.where(n3 > maxn, res / n3 * maxn, res)


_finalize = pl.pallas_call(
    _finalize_body,
    out_shape=jax.ShapeDtypeStruct((NACC, D), jnp.float32),
)


def kernel(input, edge_index):
    x = input.astype(jnp.float32)
    ei = edge_index.astype(jnp.int32)
    src = ei[0]
    dst = ei[1]
    x2, lm = _node_stats(x)
    x2f = x2.reshape(N)
    m2 = _sc_dots(x, x2f, src, dst).reshape(E)
    t = _edge_t(m2.reshape(E // 128, 128)).reshape(E)
    part = _sc_agg(lm, src, dst, t)
    part_s = _sc_segsum(src, t)
    return _finalize(part, part_s.reshape(2, NACC, 1))[:N]


# speedup vs baseline: 17.4154x; 1.0749x over previous
import jax
import jax.numpy as jnp
from jax import lax
from jax.experimental import pallas as pl
from jax.experimental.pallas import tpu as pltpu
from jax.experimental.pallas import tpu_sc as plsc

N = 10000
D = 128
E = 320000
EPS = 1e-15
BALL_EPS = 4e-3
DPAD = 144
NW = 32
EPW = E // NW
B = 80
NB = EPW // B
PAIRS = (NB - 1) // 2
GROUPS = B // 16
BSS = 16
NBSS = EPW // BSS
NACC = 10240
RPS = NACC // 16
ZROWS = 32


def _artanh(v):
    v = jnp.clip(v, -1.0 + 1e-7, 1.0 - 1e-7)
    return 0.5 * (jnp.log1p(v) - jnp.log1p(-v))


def _node_stats_body(x_ref, x2_ref, lm_ref):
    x = x_ref[...]
    x2 = jnp.sum(x * x, axis=1, keepdims=True)
    nrm = jnp.maximum(jnp.sqrt(x2), EPS)
    lm_ref[...] = x / nrm * _artanh(nrm)
    x2_ref[...] = x2


_node_stats = pl.pallas_call(
    _node_stats_body,
    out_shape=(
        jax.ShapeDtypeStruct((N, 1), jnp.float32),
        jax.ShapeDtypeStruct((N, D), jnp.float32),
    ),
)


def _sc_dots_body(x_hbm, x2_hbm, src_hbm, dst_hbm, out_hbm,
                  srcb, dstb, m2b,
                  rs0, rd0, xs0, xd0, rs1, rd1, xs1, xd1,
                  m_rs0, m_rd0, m_xs0, m_xd0, m_rs1, m_rd1, m_xs1, m_xd1):
    c = lax.axis_index("c")
    s = lax.axis_index("s")
    w = s * 2 + c
    base = pl.multiple_of(w * EPW, EPW)
    pltpu.sync_copy(src_hbm.at[pl.ds(base, EPW)], srcb)
    pltpu.sync_copy(dst_hbm.at[pl.ds(base, EPW)], dstb)

    bufs = (
        (rs0, rd0, xs0, xd0, m_rs0, m_rd0, m_xs0, m_xd0),
        (rs1, rd1, xs1, xd1, m_rs1, m_rd1, m_xs1, m_xd1),
    )

    def issue(i, bs):
        off = pl.multiple_of(i * B, B)
        pltpu.async_copy(x_hbm.at[srcb.at[pl.ds(off, B)]], bs[0], bs[4])
        pltpu.async_copy(x_hbm.at[dstb.at[pl.ds(off, B)]], bs[1], bs[5])
        pltpu.async_copy(x2_hbm.at[srcb.at[pl.ds(off, B)]], bs[2], bs[6])
        pltpu.async_copy(x2_hbm.at[dstb.at[pl.ds(off, B)]], bs[3], bs[7])

    def wait(bs):
        pltpu.make_async_copy(x_hbm.at[srcb.at[pl.ds(0, B)]], bs[0], bs[4]).wait()
        pltpu.make_async_copy(x_hbm.at[dstb.at[pl.ds(0, B)]], bs[1], bs[5]).wait()
        pltpu.make_async_copy(x2_hbm.at[srcb.at[pl.ds(0, B)]], bs[2], bs[6]).wait()
        pltpu.make_async_copy(x2_hbm.at[dstb.at[pl.ds(0, B)]], bs[3], bs[7]).wait()

    lanes = lax.iota(jnp.int32, 16)
    perms = [jnp.bitwise_xor(lanes, sh) for sh in (8, 4, 2, 1)]

    def compute(i, bs):
        rs, rd, xsb, xdb = bs[0], bs[1], bs[2], bs[3]
        off = pl.multiple_of(i * B, B)

        def group(g, carry):
            gb = pl.multiple_of(g * 16, 16)
            xy = jnp.zeros((16,), jnp.float32)
            for e in range(16):
                r = gb + e
                acc = rs[r, pl.ds(0, 16)] * rd[r, pl.ds(0, 16)]
                for cc in range(1, 8):
                    acc = acc + rs[r, pl.ds(cc * 16, 16)] * rd[r, pl.ds(cc * 16, 16)]
                for p in perms:
                    acc = acc + acc.at[p].get(mode="promise_in_bounds")
                xy = jnp.where(lanes == e, acc, xy)
            x2s = xsb[pl.ds(gb, 16)]
            x2d = xdb[pl.ds(gb, 16)]
            a1 = 1.0 - 2.0 * xy + x2d
            b1 = 1.0 - x2s
            den = jnp.maximum(1.0 - 2.0 * xy + x2s * x2d, EPS)
            num2 = jnp.maximum(a1 * a1 * x2s - 2.0 * a1 * b1 * xy + b1 * b1 * x2d, 0.0)
            m2b[pl.ds(pl.multiple_of(off + gb, 16), 16)] = num2 / (den * den)
            return carry

        lax.fori_loop(0, GROUPS, group, 0)

    issue(0, bufs[0])

    def pair(j, carry):
        i0 = j * 2
        wait(bufs[0])
        issue(i0 + 1, bufs[1])
        compute(i0, bufs[0])
        wait(bufs[1])
        issue(i0 + 2, bufs[0])
        compute(i0 + 1, bufs[1])
        return carry

    lax.fori_loop(0, PAIRS, pair, 0)
    wait(bufs[0])
    compute(NB - 1, bufs[0])
    pltpu.sync_copy(m2b, out_hbm.at[w])


_sc_dots = pl.kernel(
    _sc_dots_body,
    out_type=jax.ShapeDtypeStruct((NW, EPW), jnp.float32),
    mesh=plsc.VectorSubcoreMesh(core_axis_name="c", subcore_axis_name="s"),
    scratch_types=[
        pltpu.VMEM((EPW,), jnp.int32),
        pltpu.VMEM((EPW,), jnp.int32),
        pltpu.VMEM((EPW,), jnp.float32),
        pltpu.VMEM((B, D), jnp.float32),
        pltpu.VMEM((B, D), jnp.float32),
        pltpu.VMEM((B,), jnp.float32),
        pltpu.VMEM((B,), jnp.float32),
        pltpu.VMEM((B, D), jnp.float32),
        pltpu.VMEM((B, D), jnp.float32),
        pltpu.VMEM((B,), jnp.float32),
        pltpu.VMEM((B,), jnp.float32),
    ] + [pltpu.SemaphoreType.DMA] * 8,
)


def _edge_t_body(m2_ref, t_ref):
    m2 = jnp.maximum(m2_ref[...], 0.0)
    dd = 2.0 * _artanh(jnp.sqrt(m2))
    t_ref[...] = jnp.exp(dd * dd)


_edge_t = pl.pallas_call(
    _edge_t_body,
    out_shape=jax.ShapeDtypeStruct((E // 128, 128), jnp.float32),
)


def _sc_agg_body(lm_hbm, src_hbm, dst_hbm, t_hbm, out2d_hbm, outs_hbm,
                 tb, didx0, didx1, sidx0, sidx1, rows0, rows1, zbuf, zsbuf,
                 acc, accs,
                 m_g0, m_g1, m_i0, m_i1, m_si0, m_si1):
    c = lax.axis_index("c")
    s = lax.axis_index("s")
    w = s * 2 + c
    base = pl.multiple_of(w * EPW, EPW)

    zv = jnp.zeros((16,), jnp.float32)

    def zrow(r, carry):
        for cc in range(D // 16):
            zbuf[r, pl.ds(cc * 16, 16)] = zv
        return carry

    lax.fori_loop(0, ZROWS, zrow, 0)

    def zsrow(r, carry):
        zsbuf[pl.ds(r * 16, 16)] = zv
        return carry

    lax.fori_loop(0, RPS // 16, zsrow, 0)
    for k in range(RPS // ZROWS):
        pltpu.sync_copy(zbuf, acc.at[pl.ds(s * RPS + k * ZROWS, ZROWS)])
    pltpu.sync_copy(zsbuf, accs.at[pl.ds(s * RPS, RPS)])
    plsc.subcore_barrier()

    pltpu.sync_copy(t_hbm.at[pl.ds(base, EPW)], tb)

    didx = (didx0, didx1)
    sidx = (sidx0, sidx1)
    rows = (rows0, rows1)
    m_g = (m_g0, m_g1)
    m_i = (m_i0, m_i1)
    m_si = (m_si0, m_si1)

    def ixissue(i, b):
        off = pl.multiple_of(i * B, B)
        pltpu.async_copy(dst_hbm.at[pl.ds(base + off, B)], didx[b], m_i[b])

    def ixwait(b):
        pltpu.make_async_copy(dst_hbm.at[pl.ds(0, B)], didx[b], m_i[b]).wait()

    def sixissue(i, b):
        off = pl.multiple_of(i * B, B)
        pltpu.async_copy(src_hbm.at[pl.ds(base + off, B)], sidx[b], m_si[b])

    def sixwait(b):
        pltpu.make_async_copy(src_hbm.at[pl.ds(0, B)], sidx[b], m_si[b]).wait()

    def gissue(b):
        pltpu.async_copy(lm_hbm.at[didx[b]], rows[b], m_g[b])

    def gwait(b):
        pltpu.make_async_copy(lm_hbm.at[didx[b]], rows[b], m_g[b]).wait()

    def scale(i, b):
        off = pl.multiple_of(i * B, B)

        def group(g, carry):
            gb = pl.multiple_of(g * 16, 16)
            tvec = tb[pl.ds(pl.multiple_of(off + gb, 16), 16)]
            for e in range(16):
                tv = tvec[e]
                r = gb + e
                for cc in range(D // 16):
                    rows[b][r, pl.ds(cc * 16, 16)] = rows[b][r, pl.ds(cc * 16, 16)] * tv
            return carry

        lax.fori_loop(0, GROUPS, group, 0)

    pltpu.sync_copy(dst_hbm.at[pl.ds(base, B)], didx0)
    gissue(0)
    ixissue(1, 1)
    sixissue(0, 0)
    sixissue(1, 1)

    def substep(i, b):
        ixwait(1 - b)
        gissue(1 - b)
        gwait(b)
        ixissue(jnp.minimum(i + 2, NB - 1), b)
        scale(i, b)
        sixwait(b)
        pltpu.sync_copy(rows[b], acc.at[sidx[b]], add=True)
        off = pl.multiple_of(i * B, B)
        pltpu.sync_copy(tb.at[pl.ds(off, B)], accs.at[sidx[b]], add=True)
        sixissue(jnp.minimum(i + 2, NB - 1), b)

    def pair(j, carry):
        i0 = j * 2
        substep(i0, 0)
        substep(i0 + 1, 1)
        return carry

    lax.fori_loop(0, PAIRS, pair, 0)
    gwait(0)
    scale(NB - 1, 0)
    sixwait(0)
    pltpu.sync_copy(rows[0], acc.at[sidx[0]], add=True)
    pltpu.sync_copy(tb.at[pl.ds((NB - 1) * B, B)], accs.at[sidx[0]], add=True)
    ixwait(1)
    sixwait(1)
    plsc.subcore_barrier()
    pltpu.sync_copy(acc.at[pl.ds(s * RPS, RPS)], out2d_hbm.at[c, pl.ds(s * RPS, RPS)])
    pltpu.sync_copy(accs.at[pl.ds(s * RPS, RPS)], outs_hbm.at[c, pl.ds(s * RPS, RPS)])


_sc_agg = pl.kernel(
    _sc_agg_body,
    out_type=(
        jax.ShapeDtypeStruct((2, NACC, D), jnp.float32),
        jax.ShapeDtypeStruct((2, NACC), jnp.float32),
    ),
    mesh=plsc.VectorSubcoreMesh(core_axis_name="c", subcore_axis_name="s"),
    scratch_types=[
        pltpu.VMEM((EPW,), jnp.float32),
        pltpu.VMEM((B,), jnp.int32),
        pltpu.VMEM((B,), jnp.int32),
        pltpu.VMEM((B,), jnp.int32),
        pltpu.VMEM((B,), jnp.int32),
        pltpu.VMEM((B, D), jnp.float32),
        pltpu.VMEM((B, D), jnp.float32),
        pltpu.VMEM((ZROWS, D), jnp.float32),
        pltpu.VMEM((RPS,), jnp.float32),
        pltpu.VMEM_SHARED((NACC, D), jnp.float32),
        pltpu.VMEM_SHARED((NACC,), jnp.float32),
    ] + [pltpu.SemaphoreType.DMA] * 6,
)


def _finalize_body(p_ref, ps_ref, o_ref):
    num = p_ref[0] + p_ref[1]
    seg = ps_ref[0] + ps_ref[1]
    y = num / (seg + EPS)
    n2 = jnp.maximum(jnp.sqrt(jnp.sum(y * y, axis=1, keepdims=True)), EPS)
    res = jnp.tanh(n2) * y / n2
    n3 = jnp.maximum(jnp.sqrt(jnp.sum(res * res, axis=1, keepdims=True)), EPS)
    maxn = 1.0 - BALL_EPS
    o_ref[...] = jnp.where(n3 > maxn, res / n3 * maxn, res)


_finalize = pl.pallas_call(
    _finalize_body,
    out_shape=jax.ShapeDtypeStruct((NACC, D), jnp.float32),
)


def kernel(input, edge_index):
    x = input.astype(jnp.float32)
    ei = edge_index.astype(jnp.int32)
    src = ei[0]
    dst = ei[1]
    x2, lm = _node_stats(x)
    x2f = x2.reshape(N)
    m2 = _sc_dots(x, x2f, src, dst).reshape(E)
    t = _edge_t(m2.reshape(E // 128, 128)).reshape(E)
    part, part_s = _sc_agg(lm, src, dst, t)
    return _finalize(part, part_s.reshape(2, NACC, 1))[:N]
